# Initial kernel scaffold; baseline (speedup 1.0000x reference)
import jax
import jax.numpy as jnp
from jax.experimental import pallas as pl


def kernel(x, edge_index, edge_attr, mask, batch, W, W_ih, W_hh, b_ih, b_hh, Wl, bl):
    raise NotImplementedError("write your pallas kernel here")


# speedup vs baseline: 1.2121x; 1.2121x over previous
import functools

import jax
import jax.numpy as jnp
from jax import lax
from jax.experimental import pallas as pl
from jax.experimental.pallas import tpu as pltpu
from jax.experimental.pallas import tpu_sc as plsc

N_NODES = 10000
N_EDGES = 160000
D_FEAT = 128
OUT_CH = 1000
N_LAYERS = 3
N_GRAPHS = 64
D_OUT = 100

NP = 10240
CP = 1024
RB = 256
NRB = NP // RB
NCHUNK = NP // 1024
EP = 163840
ESL = EP // 16


WIN = 512
TROWS = 64


def _mp_body(m_hbm, src_hbm, dst_hbm, attr_hbm, offs_hbm, agg_hbm,
             ewin_s, ewin_d, ewin_a, obuf, sidx, dlw, attw, rows, acc, sem):
    cid = lax.axis_index("c")
    sid = lax.axis_index("s")
    nbw = WIN // 16

    for ci in range(NCHUNK // 2):
        c = 2 * ci + cid
        cbase = c * 1024
        rbase = cbase + sid * TROWS

        def zbody(z, _):
            for j in range(CP // 16):
                acc[z, pl.ds(j * 16, 16)] = jnp.zeros((16,), jnp.float32)
            return 0
        lax.fori_loop(0, TROWS, zbody, 0)

        k = (c * 16 + sid) * 8
        pltpu.sync_copy(offs_hbm.at[pl.ds(pl.multiple_of(k, 8), 16)], obuf)
        ov = obuf[pl.ds(0, 16)]
        e0 = ov[0]
        e1 = ov[8]
        e0a = pl.multiple_of(lax.bitwise_and(e0, ~15), 16)
        nb = (e1 - e0a + 15) // 16

        def pbody(b, _):
            par = lax.rem(b, 2)

            @pl.when(b < nb)
            def _issue():
                @pl.when(lax.rem(b, nbw) == 0)
                def _win():
                    w0 = pl.multiple_of(e0a + (b // nbw) * WIN, 16)
                    pltpu.sync_copy(src_hbm.at[pl.ds(w0, WIN)], ewin_s)
                    pltpu.sync_copy(dst_hbm.at[pl.ds(w0, WIN)], ewin_d)
                    pltpu.sync_copy(attr_hbm.at[pl.ds(w0, WIN)], ewin_a)
                base = lax.rem(b, nbw) * 16
                lane = lax.iota(jnp.int32, 16) + (e0a + b * 16)
                valid = (lane >= e0) & (lane < e1)
                svv = jnp.where(valid, ewin_s[pl.ds(base, 16)], 0)
                dlv = jnp.where(valid, ewin_d[pl.ds(base, 16)] - rbase, 0)
                avv = jnp.where(valid, ewin_a[pl.ds(base, 16)], 0.0)
                sidx[par, :] = svv
                dlw[par, :] = dlv
                attw[par, :] = avv
                pltpu.async_copy(m_hbm.at[sidx.at[par]], rows.at[par],
                                 sem.at[par])

            @pl.when(b > 0)
            def _compute():
                pp = lax.rem(b + 1, 2)
                pltpu.make_async_copy(m_hbm.at[sidx.at[pp]], rows.at[pp],
                                      sem.at[pp]).wait()
                dpv = dlw[pp, :]
                apv = attw[pp, :]
                for r in range(16):
                    sp = jnp.full((16,), apv[r], jnp.float32)
                    dl = dpv[r]
                    def cbody(jo, _3):
                        for ji in range(4):
                            sl = pl.ds(jo * 64 + ji * 16, 16)
                            acc[dl, sl] = acc[dl, sl] + rows[pp, r, sl] * sp
                        return 0
                    lax.fori_loop(0, CP // 64, cbody, 0)
            return 0
        lax.fori_loop(0, nb + 1, pbody, 0)

        pltpu.sync_copy(acc, agg_hbm.at[pl.ds(rbase, TROWS)])


@functools.cache
def _mp_build():
    return functools.partial(
        pl.kernel,
        mesh=plsc.VectorSubcoreMesh(core_axis_name="c", subcore_axis_name="s"),
        out_type=jax.ShapeDtypeStruct((NP, CP), jnp.float32),
        scratch_types=[
            pltpu.VMEM((WIN,), jnp.int32),
            pltpu.VMEM((WIN,), jnp.int32),
            pltpu.VMEM((WIN,), jnp.float32),
            pltpu.VMEM((16,), jnp.int32),
            pltpu.VMEM((2, 16), jnp.int32),
            pltpu.VMEM((2, 16), jnp.int32),
            pltpu.VMEM((2, 16), jnp.float32),
            pltpu.VMEM((2, 16, CP), jnp.float32),
            pltpu.VMEM((TROWS, CP), jnp.float32),
            pltpu.SemaphoreType.DMA((2,)),
        ],
    )(_mp_body)


def _mp_call(m, srcs, dsts, attrs, offs):
    return _mp_build()(m, srcs, dsts, attrs, offs)


def _mm_body(h_ref, w_ref, o_ref):
    o_ref[...] = jnp.dot(h_ref[...].astype(jnp.bfloat16), w_ref[...],
                         preferred_element_type=jnp.float32)


_mm_call = pl.pallas_call(
    _mm_body,
    grid=(NRB,),
    in_specs=[
        pl.BlockSpec((RB, CP), lambda i: (i, 0)),
        pl.BlockSpec((CP, CP), lambda i: (0, 0)),
    ],
    out_specs=pl.BlockSpec((RB, CP), lambda i: (i, 0)),
    out_shape=jax.ShapeDtypeStruct((NP, CP), jnp.float32),
)


def _gru_body(a_ref, h_ref, wi_ref, wh_ref, bi_ref, bh_ref, o_ref, r_s, z_s):
    g = pl.program_id(1)
    gi = jnp.dot(a_ref[...].astype(jnp.bfloat16), wi_ref[0],
                 preferred_element_type=jnp.float32) + bi_ref[0]
    gh = jnp.dot(h_ref[...].astype(jnp.bfloat16), wh_ref[0],
                 preferred_element_type=jnp.float32) + bh_ref[0]

    @pl.when(g == 0)
    def _():
        r_s[...] = jax.nn.sigmoid(gi + gh)

    @pl.when(g == 1)
    def _():
        z_s[...] = jax.nn.sigmoid(gi + gh)

    @pl.when(g == 2)
    def _():
        n = jnp.tanh(gi + r_s[...] * gh)
        z = z_s[...]
        o_ref[...] = (1.0 - z) * n + z * h_ref[...]


_gru_call = pl.pallas_call(
    _gru_body,
    grid=(NRB, 3),
    in_specs=[
        pl.BlockSpec((RB, CP), lambda i, g: (i, 0)),
        pl.BlockSpec((RB, CP), lambda i, g: (i, 0)),
        pl.BlockSpec((1, CP, CP), lambda i, g: (g, 0, 0)),
        pl.BlockSpec((1, CP, CP), lambda i, g: (g, 0, 0)),
        pl.BlockSpec((1, 1, CP), lambda i, g: (g, 0, 0)),
        pl.BlockSpec((1, 1, CP), lambda i, g: (g, 0, 0)),
    ],
    out_specs=pl.BlockSpec((RB, CP), lambda i, g: (i, 0)),
    out_shape=jax.ShapeDtypeStruct((NP, CP), jnp.float32),
    scratch_shapes=[
        pltpu.VMEM((RB, CP), jnp.float32),
        pltpu.VMEM((RB, CP), jnp.float32),
    ],
)


def _fin_body(h_ref, b3_ref, wl_ref, bls_ref, o_ref, s_acc, c_acc):
    i = pl.program_id(0)

    @pl.when(i == 0)
    def _():
        s_acc[...] = jnp.zeros_like(s_acc)
        c_acc[...] = jnp.zeros_like(c_acc)

    hb = jnp.maximum(h_ref[...], 0.0)
    s = jnp.sum(hb * wl_ref[...], axis=1)
    bv = b3_ref[0, 0, :]
    gio = lax.broadcasted_iota(jnp.int32, (RB, 128), 1)
    mask = bv[:, None] == gio
    sm = jnp.where(mask, s[:, None], 0.0)
    s_acc[...] = s_acc[...] + jnp.sum(sm.reshape(8, RB // 8, 128), axis=1)
    cm = jnp.where(mask, 1.0, 0.0)
    c_acc[...] = c_acc[...] + jnp.sum(cm.reshape(8, RB // 8, 128), axis=1)

    @pl.when(i == NRB - 1)
    def _():
        sums = jnp.sum(s_acc[...], axis=0, keepdims=True)
        cnts = jnp.sum(c_acc[...], axis=0, keepdims=True)
        vals = sums / (float(D_OUT) * jnp.maximum(cnts, 1.0)) \
            + bls_ref[0, 0] / float(D_OUT)
        o_ref[...] = jnp.where(cnts > 0, vals, 0.0)


_fin_call = pl.pallas_call(
    _fin_body,
    grid=(NRB,),
    in_specs=[
        pl.BlockSpec((RB, CP), lambda i: (i, 0)),
        pl.BlockSpec((1, 1, RB), lambda i: (i, 0, 0)),
        pl.BlockSpec((1, CP), lambda i: (0, 0)),
        pl.BlockSpec((1, 1), lambda i: (0, 0)),
    ],
    out_specs=pl.BlockSpec((1, 128), lambda i: (0, 0)),
    out_shape=jax.ShapeDtypeStruct((1, 128), jnp.float32),
    scratch_shapes=[
        pltpu.VMEM((8, 128), jnp.float32),
        pltpu.VMEM((8, 128), jnp.float32),
    ],
)


def kernel(x, edge_index, edge_attr, mask, batch, W, W_ih, W_hh, b_ih, b_hh,
           Wl, bl):
    f32 = jnp.float32
    h = jnp.zeros((NP, CP), f32).at[:N_NODES, :D_FEAT].set(x)
    bf16 = jnp.bfloat16
    Wp = jnp.zeros((N_LAYERS, CP, CP), f32).at[:, :OUT_CH, :OUT_CH].set(
        W).astype(bf16)
    A_ih = jnp.zeros((3, CP, CP), f32).at[:, :OUT_CH, :OUT_CH].set(
        jnp.transpose(W_ih.reshape(3, OUT_CH, OUT_CH), (0, 2, 1))).astype(bf16)
    A_hh = jnp.zeros((3, CP, CP), f32).at[:, :OUT_CH, :OUT_CH].set(
        jnp.transpose(W_hh.reshape(3, OUT_CH, OUT_CH), (0, 2, 1))).astype(bf16)
    B_ih = jnp.zeros((3, 1, CP), f32).at[:, 0, :OUT_CH].set(
        b_ih.reshape(3, OUT_CH))
    B_hh = jnp.zeros((3, 1, CP), f32).at[:, 0, :OUT_CH].set(
        b_hh.reshape(3, OUT_CH))
    wl_sum = jnp.zeros((1, CP), f32).at[0, :OUT_CH].set(jnp.sum(Wl, axis=0))
    bl_sum = jnp.reshape(jnp.sum(bl), (1, 1))

    order = jnp.argsort(edge_index[1])
    srcs = jnp.pad(edge_index[0][order], (0, EP - N_EDGES))
    dsts = jnp.pad(edge_index[1][order], (0, EP - N_EDGES),
                   constant_values=NP - 1)
    attrs = jnp.pad(edge_attr[order], (0, EP - N_EDGES))
    offs64 = jnp.searchsorted(
        dsts[:N_EDGES], jnp.arange(0, NP + TROWS, TROWS, dtype=jnp.int32)
    ).astype(jnp.int32)
    offs = jnp.repeat(offs64, 8)
    batch_p = jnp.pad(batch, (0, NP - N_NODES), constant_values=127)
    batch3 = batch_p.reshape(NRB, 1, RB)

    for i in range(N_LAYERS):
        m = _mm_call(h, Wp[i])
        agg = _mp_call(m, srcs, dsts, attrs, offs)
        h = _gru_call(agg, h, A_ih, A_hh, B_ih, B_hh)

    out = _fin_call(h, batch3, wl_sum, bl_sum)
    return out[0, :N_GRAPHS]


# speedup vs baseline: 1.3258x; 1.0938x over previous
import functools

import jax
import jax.numpy as jnp
from jax import lax
from jax.experimental import pallas as pl
from jax.experimental.pallas import tpu as pltpu
from jax.experimental.pallas import tpu_sc as plsc

N_NODES = 10000
N_EDGES = 160000
D_FEAT = 128
OUT_CH = 1000
N_LAYERS = 3
N_GRAPHS = 64
D_OUT = 100

NP = 10240
CP = 1024
RB = 256
NRB = NP // RB
NCHUNK = NP // 1024
EP = 163840
ESL = EP // 16


WIN = 512
TROWS = 64


def _mp_body(m_hbm, src_hbm, dst_hbm, attr_hbm, offs_hbm, agg_hbm,
             ewin_s, ewin_d, ewin_a, obuf, sidx, dlw, attw, rows, acc, sem):
    cid = lax.axis_index("c")
    sid = lax.axis_index("s")
    nbw = WIN // 16

    for ci in range(NCHUNK // 2):
        c = 2 * ci + cid
        cbase = c * 1024
        rbase = cbase + sid * TROWS

        def zbody(z, _):
            for j in range(CP // 16):
                acc[z, pl.ds(j * 16, 16)] = jnp.zeros((16,), jnp.float32)
            return 0
        lax.fori_loop(0, TROWS, zbody, 0)

        k = (c * 16 + sid) * 8
        pltpu.sync_copy(offs_hbm.at[pl.ds(pl.multiple_of(k, 8), 16)], obuf)
        ov = obuf[pl.ds(0, 16)]
        e0 = ov[0]
        e1 = ov[8]
        e0a = pl.multiple_of(lax.bitwise_and(e0, ~15), 16)
        nb = (e1 - e0a + 15) // 16

        def pbody(b, _):
            par = lax.rem(b, 2)

            @pl.when(b < nb)
            def _issue():
                @pl.when(lax.rem(b, nbw) == 0)
                def _win():
                    w0 = pl.multiple_of(e0a + (b // nbw) * WIN, 16)
                    pltpu.sync_copy(src_hbm.at[pl.ds(w0, WIN)], ewin_s)
                    pltpu.sync_copy(dst_hbm.at[pl.ds(w0, WIN)], ewin_d)
                    pltpu.sync_copy(attr_hbm.at[pl.ds(w0, WIN)], ewin_a)
                base = lax.rem(b, nbw) * 16
                lane = lax.iota(jnp.int32, 16) + (e0a + b * 16)
                valid = (lane >= e0) & (lane < e1)
                svv = jnp.where(valid, ewin_s[pl.ds(base, 16)], 0)
                dlv = jnp.where(valid, ewin_d[pl.ds(base, 16)] - rbase, 0)
                avv = jnp.where(valid, ewin_a[pl.ds(base, 16)], 0.0)
                sidx[par, :] = svv
                dlw[par, :] = dlv
                attw[par, :] = avv
                pltpu.async_copy(m_hbm.at[sidx.at[par]], rows.at[par],
                                 sem.at[par])

            @pl.when(b > 0)
            def _compute():
                pp = lax.rem(b + 1, 2)
                pltpu.make_async_copy(m_hbm.at[sidx.at[pp]], rows.at[pp],
                                      sem.at[pp]).wait()
                dpv = dlw[pp, :]
                apv = attw[pp, :]
                same = dpv[0] == dpv[15]

                @pl.when(same)
                def _fast():
                    dl0 = dpv[0]
                    sps = [jnp.full((16,), apv[r], jnp.float32)
                           for r in range(16)]
                    def fj(j, _3):
                        sl = pl.ds(j * 16, 16)
                        s = rows[pp, 0, sl] * sps[0]
                        for r in range(1, 16):
                            s = s + rows[pp, r, sl] * sps[r]
                        acc[dl0, sl] = acc[dl0, sl] + s
                        return 0
                    lax.fori_loop(0, CP // 16, fj, 0)

                @pl.when(jnp.logical_not(same))
                def _slow():
                    for r in range(16):
                        sp = jnp.full((16,), apv[r], jnp.float32)
                        dl = dpv[r]
                        def cbody(jo, _3):
                            for ji in range(2):
                                sl = pl.ds(jo * 32 + ji * 16, 16)
                                acc[dl, sl] = acc[dl, sl] + rows[pp, r, sl] * sp
                            return 0
                        lax.fori_loop(0, CP // 32, cbody, 0)
            return 0
        lax.fori_loop(0, nb + 1, pbody, 0)

        pltpu.sync_copy(acc, agg_hbm.at[pl.ds(rbase, TROWS)])


@functools.cache
def _mp_build():
    return functools.partial(
        pl.kernel,
        mesh=plsc.VectorSubcoreMesh(core_axis_name="c", subcore_axis_name="s"),
        out_type=jax.ShapeDtypeStruct((NP, CP), jnp.float32),
        scratch_types=[
            pltpu.VMEM((WIN,), jnp.int32),
            pltpu.VMEM((WIN,), jnp.int32),
            pltpu.VMEM((WIN,), jnp.float32),
            pltpu.VMEM((16,), jnp.int32),
            pltpu.VMEM((2, 16), jnp.int32),
            pltpu.VMEM((2, 16), jnp.int32),
            pltpu.VMEM((2, 16), jnp.float32),
            pltpu.VMEM((2, 16, CP), jnp.float32),
            pltpu.VMEM((TROWS, CP), jnp.float32),
            pltpu.SemaphoreType.DMA((2,)),
        ],
    )(_mp_body)


def _mp_call(m, srcs, dsts, attrs, offs):
    return _mp_build()(m, srcs, dsts, attrs, offs)


def _mm_body(h_ref, w_ref, o_ref):
    o_ref[...] = jnp.dot(h_ref[...].astype(jnp.bfloat16), w_ref[...],
                         preferred_element_type=jnp.float32)


_mm_call = pl.pallas_call(
    _mm_body,
    grid=(NRB,),
    in_specs=[
        pl.BlockSpec((RB, CP), lambda i: (i, 0)),
        pl.BlockSpec((CP, CP), lambda i: (0, 0)),
    ],
    out_specs=pl.BlockSpec((RB, CP), lambda i: (i, 0)),
    out_shape=jax.ShapeDtypeStruct((NP, CP), jnp.float32),
)


def _gru_body(a_ref, h_ref, wi_ref, wh_ref, bi_ref, bh_ref, o_ref, r_s, z_s):
    g = pl.program_id(1)
    gi = jnp.dot(a_ref[...].astype(jnp.bfloat16), wi_ref[0],
                 preferred_element_type=jnp.float32) + bi_ref[0]
    gh = jnp.dot(h_ref[...].astype(jnp.bfloat16), wh_ref[0],
                 preferred_element_type=jnp.float32) + bh_ref[0]

    @pl.when(g == 0)
    def _():
        r_s[...] = jax.nn.sigmoid(gi + gh)

    @pl.when(g == 1)
    def _():
        z_s[...] = jax.nn.sigmoid(gi + gh)

    @pl.when(g == 2)
    def _():
        n = jnp.tanh(gi + r_s[...] * gh)
        z = z_s[...]
        o_ref[...] = (1.0 - z) * n + z * h_ref[...]


_gru_call = pl.pallas_call(
    _gru_body,
    grid=(NRB, 3),
    in_specs=[
        pl.BlockSpec((RB, CP), lambda i, g: (i, 0)),
        pl.BlockSpec((RB, CP), lambda i, g: (i, 0)),
        pl.BlockSpec((1, CP, CP), lambda i, g: (g, 0, 0)),
        pl.BlockSpec((1, CP, CP), lambda i, g: (g, 0, 0)),
        pl.BlockSpec((1, 1, CP), lambda i, g: (g, 0, 0)),
        pl.BlockSpec((1, 1, CP), lambda i, g: (g, 0, 0)),
    ],
    out_specs=pl.BlockSpec((RB, CP), lambda i, g: (i, 0)),
    out_shape=jax.ShapeDtypeStruct((NP, CP), jnp.float32),
    scratch_shapes=[
        pltpu.VMEM((RB, CP), jnp.float32),
        pltpu.VMEM((RB, CP), jnp.float32),
    ],
)


def _fin_body(h_ref, b3_ref, wl_ref, bls_ref, o_ref, s_acc, c_acc):
    i = pl.program_id(0)

    @pl.when(i == 0)
    def _():
        s_acc[...] = jnp.zeros_like(s_acc)
        c_acc[...] = jnp.zeros_like(c_acc)

    hb = jnp.maximum(h_ref[...], 0.0)
    s = jnp.sum(hb * wl_ref[...], axis=1)
    bv = b3_ref[0, 0, :]
    gio = lax.broadcasted_iota(jnp.int32, (RB, 128), 1)
    mask = bv[:, None] == gio
    sm = jnp.where(mask, s[:, None], 0.0)
    s_acc[...] = s_acc[...] + jnp.sum(sm.reshape(8, RB // 8, 128), axis=1)
    cm = jnp.where(mask, 1.0, 0.0)
    c_acc[...] = c_acc[...] + jnp.sum(cm.reshape(8, RB // 8, 128), axis=1)

    @pl.when(i == NRB - 1)
    def _():
        sums = jnp.sum(s_acc[...], axis=0, keepdims=True)
        cnts = jnp.sum(c_acc[...], axis=0, keepdims=True)
        vals = sums / (float(D_OUT) * jnp.maximum(cnts, 1.0)) \
            + bls_ref[0, 0] / float(D_OUT)
        o_ref[...] = jnp.where(cnts > 0, vals, 0.0)


_fin_call = pl.pallas_call(
    _fin_body,
    grid=(NRB,),
    in_specs=[
        pl.BlockSpec((RB, CP), lambda i: (i, 0)),
        pl.BlockSpec((1, 1, RB), lambda i: (i, 0, 0)),
        pl.BlockSpec((1, CP), lambda i: (0, 0)),
        pl.BlockSpec((1, 1), lambda i: (0, 0)),
    ],
    out_specs=pl.BlockSpec((1, 128), lambda i: (0, 0)),
    out_shape=jax.ShapeDtypeStruct((1, 128), jnp.float32),
    scratch_shapes=[
        pltpu.VMEM((8, 128), jnp.float32),
        pltpu.VMEM((8, 128), jnp.float32),
    ],
)


def kernel(x, edge_index, edge_attr, mask, batch, W, W_ih, W_hh, b_ih, b_hh,
           Wl, bl):
    f32 = jnp.float32
    h = jnp.zeros((NP, CP), f32).at[:N_NODES, :D_FEAT].set(x)
    bf16 = jnp.bfloat16
    Wp = jnp.zeros((N_LAYERS, CP, CP), f32).at[:, :OUT_CH, :OUT_CH].set(
        W).astype(bf16)
    A_ih = jnp.zeros((3, CP, CP), f32).at[:, :OUT_CH, :OUT_CH].set(
        jnp.transpose(W_ih.reshape(3, OUT_CH, OUT_CH), (0, 2, 1))).astype(bf16)
    A_hh = jnp.zeros((3, CP, CP), f32).at[:, :OUT_CH, :OUT_CH].set(
        jnp.transpose(W_hh.reshape(3, OUT_CH, OUT_CH), (0, 2, 1))).astype(bf16)
    B_ih = jnp.zeros((3, 1, CP), f32).at[:, 0, :OUT_CH].set(
        b_ih.reshape(3, OUT_CH))
    B_hh = jnp.zeros((3, 1, CP), f32).at[:, 0, :OUT_CH].set(
        b_hh.reshape(3, OUT_CH))
    wl_sum = jnp.zeros((1, CP), f32).at[0, :OUT_CH].set(jnp.sum(Wl, axis=0))
    bl_sum = jnp.reshape(jnp.sum(bl), (1, 1))

    order = jnp.argsort(edge_index[1])
    srcs = jnp.pad(edge_index[0][order], (0, EP - N_EDGES))
    dsts = jnp.pad(edge_index[1][order], (0, EP - N_EDGES),
                   constant_values=NP - 1)
    attrs = jnp.pad(edge_attr[order], (0, EP - N_EDGES))
    offs64 = jnp.searchsorted(
        dsts[:N_EDGES], jnp.arange(0, NP + TROWS, TROWS, dtype=jnp.int32)
    ).astype(jnp.int32)
    offs = jnp.repeat(offs64, 8)
    batch_p = jnp.pad(batch, (0, NP - N_NODES), constant_values=127)
    batch3 = batch_p.reshape(NRB, 1, RB)

    for i in range(N_LAYERS):
        m = _mm_call(h, Wp[i])
        agg = _mp_call(m, srcs, dsts, attrs, offs)
        h = _gru_call(agg, h, A_ih, A_hh, B_ih, B_hh)

    out = _fin_call(h, batch3, wl_sum, bl_sum)
    return out[0, :N_GRAPHS]
